# Initial kernel scaffold; baseline (speedup 1.0000x reference)
import jax
import jax.numpy as jnp
from jax.experimental import pallas as pl


def kernel(feature, label, centers):
    raise NotImplementedError("write your pallas kernel here")


# speedup vs baseline: 1.8081x; 1.8081x over previous
import functools

import jax
import jax.numpy as jnp
from jax import lax
from jax.experimental import pallas as pl
from jax.experimental.pallas import tpu as pltpu
from jax.experimental.pallas import tpu_sc as plsc

EPS = 1e-8
NUM_CLASSES = 150
CHANNELS = 192
BATCH = 8
HW = 128 * 128
N_PIX = BATCH * HW

NC = 2
NS = 16
NW = NC * NS
LANES = 16

K_PAD = 160
CPT = CHANNELS // NW
ROWS_PER_TILE = (BATCH * 128) // NS
CNT_SLICE = N_PIX // NW


def _sc_segment_sums(label, feat3):
    mesh = plsc.VectorSubcoreMesh(core_axis_name="c", subcore_axis_name="s")

    @functools.partial(
        pl.kernel,
        out_type=(
            jax.ShapeDtypeStruct((CHANNELS, K_PAD), jnp.float32),
            jax.ShapeDtypeStruct((NW, K_PAD), jnp.float32),
        ),
        mesh=mesh,
        compiler_params=pltpu.CompilerParams(use_tc_tiling_on_sc=False,
                                             needs_layout_passes=False),
        scratch_types=[
            pltpu.VMEM((512,), jnp.int32),
            pltpu.VMEM((ROWS_PER_TILE * 128,), jnp.int32),
            pltpu.VMEM_SHARED((N_PIX,), jnp.int32),
            pltpu.VMEM((HW,), jnp.int32),
            pltpu.VMEM((HW,), jnp.float32),
            pltpu.VMEM((CPT, LANES * K_PAD), jnp.float32),
            pltpu.VMEM((CPT, K_PAD), jnp.float32),
            pltpu.VMEM((LANES * K_PAD,), jnp.float32),
            pltpu.VMEM((K_PAD,), jnp.float32),
        ],
    )
    def sc_main(label_hbm, feat_hbm, sum_out, cnt_out,
                row_buf, lab_stage, lab_shared, lab_tile, feat_buf,
                acc, acc_out, cnt_acc, cnt_vec):
        cid = lax.axis_index("c")
        sid = lax.axis_index("s")
        gwid = cid * NS + sid

        zeros16 = jnp.zeros((LANES,), jnp.float32)
        ones16 = jnp.ones((LANES,), jnp.float32)
        iota16 = lax.iota(jnp.int32, LANES)
        lane_off = iota16 * K_PAD

        b0 = sid // 2
        r0 = (sid % 2) * ROWS_PER_TILE

        @pl.loop(0, ROWS_PER_TILE)
        def _down(r):
            pltpu.sync_copy(label_hbm.at[b0, (r0 + r) * 4], row_buf)
            for j in range(8):
                ix = iota16 * 4 + (j * 64)
                v = plsc.load_gather(row_buf, [ix])
                lab_stage[pl.ds(r * 128 + j * LANES, LANES)] = v + lane_off

        pltpu.sync_copy(
            lab_stage, lab_shared.at[pl.ds(sid * (ROWS_PER_TILE * 128),
                                           ROWS_PER_TILE * 128)])
        plsc.subcore_barrier()

        @pl.loop(0, (LANES * K_PAD) // LANES)
        def _zc(i):
            cnt_acc[pl.ds(i * LANES, LANES)] = zeros16

        pltpu.sync_copy(lab_shared.at[pl.ds(gwid * CNT_SLICE, CNT_SLICE)],
                        lab_tile.at[pl.ds(0, CNT_SLICE)])

        @pl.loop(0, CNT_SLICE // LANES)
        def _cnt(i):
            ix = lab_tile[pl.ds(i * LANES, LANES)]
            plsc.addupdate_scatter(cnt_acc, [ix], ones16)

        @pl.loop(0, K_PAD // LANES)
        def _credu(i):
            tot = zeros16
            for l in range(LANES):
                tot = tot + cnt_acc[pl.ds(l * K_PAD + i * LANES, LANES)]
            cnt_vec[pl.ds(i * LANES, LANES)] = tot

        pltpu.sync_copy(cnt_vec, cnt_out.at[gwid])

        ch_base = gwid * CPT
        for cl in range(CPT):
            accl = acc.at[cl]

            @pl.loop(0, (LANES * K_PAD) // LANES)
            def _za(i, accl=accl):
                accl[pl.ds(i * LANES, LANES)] = zeros16

        for b in range(BATCH):
            pltpu.sync_copy(lab_shared.at[pl.ds(b * HW, HW)], lab_tile)
            for cl in range(CPT):
                pltpu.sync_copy(feat_hbm.at[b, ch_base + cl], feat_buf)
                accl = acc.at[cl]

                @pl.loop(0, HW // LANES)
                def _seg(i, accl=accl):
                    ix = lab_tile[pl.ds(i * LANES, LANES)]
                    v = feat_buf[pl.ds(i * LANES, LANES)]
                    plsc.addupdate_scatter(accl, [ix], v)

        for cl in range(CPT):
            accl = acc.at[cl]
            outl = acc_out.at[cl]

            @pl.loop(0, K_PAD // LANES)
            def _aredu(i, accl=accl, outl=outl):
                tot = zeros16
                for l in range(LANES):
                    tot = tot + accl[pl.ds(l * K_PAD + i * LANES, LANES)]
                outl[pl.ds(i * LANES, LANES)] = tot

        pltpu.sync_copy(acc_out, sum_out.at[pl.ds(ch_base, CPT)])

    return sc_main(label, feat3)


def _epilogue(sum_pad, cnt_pad, cenT_pad):
    def body(sum_ref, cnt_ref, cen_ref, out_ref):
        cnt = jnp.sum(cnt_ref[...], axis=0, keepdims=True)
        present = cnt > 0.0
        cnt_safe = jnp.where(present, cnt, 1.0)
        mean = sum_ref[...] / cnt_safe
        cT = cen_ref[...]
        dot = jnp.sum(mean * cT, axis=0, keepdims=True)
        n1 = jnp.maximum(jnp.sqrt(jnp.sum(mean * mean, axis=0, keepdims=True)), EPS)
        n2 = jnp.maximum(jnp.sqrt(jnp.sum(cT * cT, axis=0, keepdims=True)), EPS)
        cos = jnp.where(present, dot / (n1 * n2), 0.0)
        npres = jnp.sum(present.astype(jnp.float32))
        loss = 1.0 - jnp.sum(cos) / npres
        out_ref[...] = loss[None, None]

    return pl.pallas_call(
        body,
        out_shape=jax.ShapeDtypeStruct((1, 1), jnp.float32),
    )(sum_pad, cnt_pad, cenT_pad)


def kernel(feature, label, centers):
    feat3 = feature.reshape(BATCH, CHANNELS, HW)
    if label.ndim == 4:
        label = label[:, 0]
    sum_pad, cnt_pad = _sc_segment_sums(label, feat3)
    cenT_pad = jnp.pad(centers.T, ((0, 0), (0, K_PAD - NUM_CLASSES)))
    loss = _epilogue(sum_pad, cnt_pad, cenT_pad)
    return loss[0, 0]


# speedup vs baseline: 2.3221x; 1.2843x over previous
import functools

import jax
import jax.numpy as jnp
from jax import lax
from jax.experimental import pallas as pl
from jax.experimental.pallas import tpu as pltpu
from jax.experimental.pallas import tpu_sc as plsc

EPS = 1e-8
NUM_CLASSES = 150
CHANNELS = 192
BATCH = 8
HW = 128 * 128
N_PIX = BATCH * HW

NC = 2
NS = 16
NW = NC * NS
LANES = 16

K_PAD = 160
CPT = CHANNELS // NW
ROWS_PER_TILE = (BATCH * 128) // NS
CNT_SLICE = N_PIX // NW


def _sc_segment_sums(label, feat3):
    mesh = plsc.VectorSubcoreMesh(core_axis_name="c", subcore_axis_name="s")

    @functools.partial(
        pl.kernel,
        out_type=(
            jax.ShapeDtypeStruct((CHANNELS, K_PAD), jnp.float32),
            jax.ShapeDtypeStruct((NW, K_PAD), jnp.float32),
        ),
        mesh=mesh,
        compiler_params=pltpu.CompilerParams(use_tc_tiling_on_sc=False,
                                             needs_layout_passes=False),
        scratch_types=[
            pltpu.VMEM((64, 512), jnp.int32),
            pltpu.VMEM((ROWS_PER_TILE * 128,), jnp.int32),
            pltpu.VMEM_SHARED((N_PIX,), jnp.int32),
            pltpu.VMEM((HW,), jnp.int32),
            pltpu.VMEM((HW,), jnp.float32),
            pltpu.VMEM((HW,), jnp.float32),
            pltpu.VMEM((CPT, LANES * K_PAD), jnp.float32),
            pltpu.VMEM((CPT, K_PAD), jnp.float32),
            pltpu.VMEM((LANES * K_PAD,), jnp.float32),
            pltpu.VMEM((K_PAD,), jnp.float32),
            pltpu.SemaphoreType.DMA,
            pltpu.SemaphoreType.DMA,
        ],
    )
    def sc_main(label_hbm, feat_hbm, sum_out, cnt_out,
                row_blk, lab_stage, lab_shared, lab_tile, feat_buf0, feat_buf1,
                acc, acc_out, cnt_acc, cnt_vec, sem0, sem1):
        cid = lax.axis_index("c")
        sid = lax.axis_index("s")
        gwid = cid * NS + sid

        zeros16 = jnp.zeros((LANES,), jnp.float32)
        ones16 = jnp.ones((LANES,), jnp.float32)
        iota16 = lax.iota(jnp.int32, LANES)
        lane_off = iota16 * K_PAD

        b0 = sid // 2
        r0 = (sid % 2) * ROWS_PER_TILE
        for chunk in range(4):
            pltpu.sync_copy(label_hbm.at[b0, pl.ds((r0 + chunk * 16) * 4, 64)],
                            row_blk)

            @pl.loop(0, 16, unroll=2)
            def _down(rr, chunk=chunk):
                rids = jnp.broadcast_to(rr * 4, (LANES,)).astype(jnp.int32)
                for j in range(8):
                    cix = iota16 * 4 + (j * 64)
                    v = plsc.load_gather(row_blk, [rids, cix])
                    lab_stage[pl.ds((chunk * 16 + rr) * 128 + j * LANES,
                                    LANES)] = v + lane_off

        pltpu.sync_copy(
            lab_stage, lab_shared.at[pl.ds(sid * (ROWS_PER_TILE * 128),
                                           ROWS_PER_TILE * 128)])
        plsc.subcore_barrier()

        @pl.loop(0, (LANES * K_PAD) // LANES, unroll=8)
        def _zc(i):
            cnt_acc[pl.ds(i * LANES, LANES)] = zeros16

        pltpu.sync_copy(lab_shared.at[pl.ds(gwid * CNT_SLICE, CNT_SLICE)],
                        lab_tile.at[pl.ds(0, CNT_SLICE)])

        @pl.loop(0, CNT_SLICE // LANES, unroll=8)
        def _cnt(i):
            ix = lab_tile[pl.ds(i * LANES, LANES)]
            plsc.addupdate_scatter(cnt_acc, [ix], ones16)

        @pl.loop(0, K_PAD // LANES)
        def _credu(i):
            tot = zeros16
            for l in range(LANES):
                tot = tot + cnt_acc[pl.ds(l * K_PAD + i * LANES, LANES)]
            cnt_vec[pl.ds(i * LANES, LANES)] = tot

        pltpu.sync_copy(cnt_vec, cnt_out.at[gwid])

        ch_base = gwid * CPT
        for cl in range(CPT):
            accl = acc.at[cl]

            @pl.loop(0, (LANES * K_PAD) // LANES, unroll=8)
            def _za(i, accl=accl):
                accl[pl.ds(i * LANES, LANES)] = zeros16

        feat_bufs = (feat_buf0, feat_buf1)
        sems = (sem0, sem1)
        n_planes = BATCH * CPT

        def _start(p):
            b, cl = divmod(p, CPT)
            return pltpu.async_copy(feat_hbm.at[b, ch_base + cl],
                                    feat_bufs[p % 2], sems[p % 2])

        descs = [_start(0), None]
        for p in range(n_planes):
            b, cl = divmod(p, CPT)
            if cl == 0:
                pltpu.sync_copy(lab_shared.at[pl.ds(b * HW, HW)], lab_tile)
            if p + 1 < n_planes:
                descs[(p + 1) % 2] = _start(p + 1)
            descs[p % 2].wait()
            fb = feat_bufs[p % 2]
            accl = acc.at[cl]

            @pl.loop(0, HW // LANES, unroll=8)
            def _seg(i, accl=accl, fb=fb):
                ix = lab_tile[pl.ds(i * LANES, LANES)]
                v = fb[pl.ds(i * LANES, LANES)]
                plsc.addupdate_scatter(accl, [ix], v)

        for cl in range(CPT):
            accl = acc.at[cl]
            outl = acc_out.at[cl]

            @pl.loop(0, K_PAD // LANES)
            def _aredu(i, accl=accl, outl=outl):
                tot = zeros16
                for l in range(LANES):
                    tot = tot + accl[pl.ds(l * K_PAD + i * LANES, LANES)]
                outl[pl.ds(i * LANES, LANES)] = tot

        pltpu.sync_copy(acc_out, sum_out.at[pl.ds(ch_base, CPT)])

    return sc_main(label, feat3)


def _epilogue(sum_pad, cnt_pad, cenT_pad):
    def body(sum_ref, cnt_ref, cen_ref, out_ref):
        cnt = jnp.sum(cnt_ref[...], axis=0, keepdims=True)
        present = cnt > 0.0
        cnt_safe = jnp.where(present, cnt, 1.0)
        mean = sum_ref[...] / cnt_safe
        cT = cen_ref[...]
        dot = jnp.sum(mean * cT, axis=0, keepdims=True)
        n1 = jnp.maximum(jnp.sqrt(jnp.sum(mean * mean, axis=0, keepdims=True)), EPS)
        n2 = jnp.maximum(jnp.sqrt(jnp.sum(cT * cT, axis=0, keepdims=True)), EPS)
        cos = jnp.where(present, dot / (n1 * n2), 0.0)
        npres = jnp.sum(present.astype(jnp.float32))
        loss = 1.0 - jnp.sum(cos) / npres
        out_ref[...] = loss[None, None]

    return pl.pallas_call(
        body,
        out_shape=jax.ShapeDtypeStruct((1, 1), jnp.float32),
    )(sum_pad, cnt_pad, cenT_pad)


def kernel(feature, label, centers):
    feat3 = feature.reshape(BATCH, CHANNELS, HW)
    if label.ndim == 4:
        label = label[:, 0]
    sum_pad, cnt_pad = _sc_segment_sums(label, feat3)
    cenT_pad = jnp.pad(centers.T, ((0, 0), (0, K_PAD - NUM_CLASSES)))
    loss = _epilogue(sum_pad, cnt_pad, cenT_pad)
    return loss[0, 0]


# speedup vs baseline: 4.4272x; 1.9065x over previous
import functools

import jax
import jax.numpy as jnp
from jax import lax
from jax.experimental import pallas as pl
from jax.experimental.pallas import tpu as pltpu
from jax.experimental.pallas import tpu_sc as plsc

EPS = 1e-8
NUM_CLASSES = 150
CHANNELS = 192
BATCH = 8
HW = 128 * 128
N_PIX = BATCH * HW

NC = 2
NS = 16
NW = NC * NS
LANES = 16

K_PAD = 160
CPT = CHANNELS // NW
ROWS_PER_TILE = (BATCH * 128) // NS
CNT_SLICE = N_PIX // NW


def _sc_segment_sums(label, feat3):
    mesh = plsc.VectorSubcoreMesh(core_axis_name="c", subcore_axis_name="s")

    @functools.partial(
        pl.kernel,
        out_type=(
            jax.ShapeDtypeStruct((CHANNELS, K_PAD), jnp.float32),
            jax.ShapeDtypeStruct((NW, K_PAD), jnp.float32),
        ),
        mesh=mesh,
        compiler_params=pltpu.CompilerParams(use_tc_tiling_on_sc=False,
                                             needs_layout_passes=False),
        scratch_types=[
            pltpu.VMEM((64, 512), jnp.int32),
            pltpu.VMEM((ROWS_PER_TILE * 128,), jnp.int32),
            pltpu.VMEM_SHARED((N_PIX,), jnp.int32),
            pltpu.VMEM((HW,), jnp.int32),
            pltpu.VMEM((HW,), jnp.float32),
            pltpu.VMEM((HW,), jnp.float32),
            pltpu.VMEM((CPT, LANES * K_PAD), jnp.float32),
            pltpu.VMEM((CPT, K_PAD), jnp.float32),
            pltpu.VMEM((LANES * K_PAD,), jnp.float32),
            pltpu.VMEM((K_PAD,), jnp.float32),
            pltpu.SemaphoreType.DMA,
            pltpu.SemaphoreType.DMA,
        ],
    )
    def sc_main(label_hbm, feat_hbm, sum_out, cnt_out,
                row_blk, lab_stage, lab_shared, lab_tile, feat_buf0, feat_buf1,
                acc, acc_out, cnt_acc, cnt_vec, sem0, sem1):
        cid = lax.axis_index("c")
        sid = lax.axis_index("s")
        gwid = cid * NS + sid

        zeros16 = jnp.zeros((LANES,), jnp.float32)
        ones16 = jnp.ones((LANES,), jnp.float32)
        iota16 = lax.iota(jnp.int32, LANES)
        lane_off = iota16 * K_PAD

        b0 = sid // 2
        r0 = (sid % 2) * ROWS_PER_TILE
        for chunk in range(4):
            pltpu.sync_copy(label_hbm.at[b0, pl.ds((r0 + chunk * 16) * 4, 64)],
                            row_blk)

            @plsc.parallel_loop(0, 16, unroll=2)
            def _down(rr, chunk=chunk):
                rids = jnp.broadcast_to(rr * 4, (LANES,)).astype(jnp.int32)
                for j in range(8):
                    cix = iota16 * 4 + (j * 64)
                    v = plsc.load_gather(row_blk, [rids, cix])
                    lab_stage[pl.ds((chunk * 16 + rr) * 128 + j * LANES,
                                    LANES)] = v + lane_off

        pltpu.sync_copy(
            lab_stage, lab_shared.at[pl.ds(sid * (ROWS_PER_TILE * 128),
                                           ROWS_PER_TILE * 128)])
        plsc.subcore_barrier()

        @pl.loop(0, (LANES * K_PAD) // LANES, unroll=8)
        def _zc(i):
            cnt_acc[pl.ds(i * LANES, LANES)] = zeros16

        pltpu.sync_copy(lab_shared.at[pl.ds(gwid * CNT_SLICE, CNT_SLICE)],
                        lab_tile.at[pl.ds(0, CNT_SLICE)])

        @plsc.parallel_loop(0, CNT_SLICE // LANES, unroll=8)
        def _cnt(i):
            ix = lab_tile[pl.ds(i * LANES, LANES)]
            plsc.addupdate_scatter(cnt_acc, [ix], ones16)

        @pl.loop(0, K_PAD // LANES)
        def _credu(i):
            tot = zeros16
            for l in range(LANES):
                tot = tot + cnt_acc[pl.ds(l * K_PAD + i * LANES, LANES)]
            cnt_vec[pl.ds(i * LANES, LANES)] = tot

        pltpu.sync_copy(cnt_vec, cnt_out.at[gwid])

        ch_base = gwid * CPT
        for cl in range(CPT):
            accl = acc.at[cl]

            @pl.loop(0, (LANES * K_PAD) // LANES, unroll=8)
            def _za(i, accl=accl):
                accl[pl.ds(i * LANES, LANES)] = zeros16

        feat_bufs = (feat_buf0, feat_buf1)
        sems = (sem0, sem1)
        n_planes = BATCH * CPT

        def _start(p):
            b, cl = divmod(p, CPT)
            return pltpu.async_copy(feat_hbm.at[b, ch_base + cl],
                                    feat_bufs[p % 2], sems[p % 2])

        descs = [_start(0), None]
        for p in range(n_planes):
            b, cl = divmod(p, CPT)
            if cl == 0:
                pltpu.sync_copy(lab_shared.at[pl.ds(b * HW, HW)], lab_tile)
            if p + 1 < n_planes:
                descs[(p + 1) % 2] = _start(p + 1)
            descs[p % 2].wait()
            fb = feat_bufs[p % 2]
            accl = acc.at[cl]

            @plsc.parallel_loop(0, HW // LANES, unroll=8)
            def _seg(i, accl=accl, fb=fb):
                ix = lab_tile[pl.ds(i * LANES, LANES)]
                v = fb[pl.ds(i * LANES, LANES)]
                plsc.addupdate_scatter(accl, [ix], v)

        for cl in range(CPT):
            accl = acc.at[cl]
            outl = acc_out.at[cl]

            @pl.loop(0, K_PAD // LANES)
            def _aredu(i, accl=accl, outl=outl):
                tot = zeros16
                for l in range(LANES):
                    tot = tot + accl[pl.ds(l * K_PAD + i * LANES, LANES)]
                outl[pl.ds(i * LANES, LANES)] = tot

        pltpu.sync_copy(acc_out, sum_out.at[pl.ds(ch_base, CPT)])

    return sc_main(label, feat3)


def _epilogue(sum_pad, cnt_pad, cenT_pad):
    def body(sum_ref, cnt_ref, cen_ref, out_ref):
        cnt = jnp.sum(cnt_ref[...], axis=0, keepdims=True)
        present = cnt > 0.0
        cnt_safe = jnp.where(present, cnt, 1.0)
        mean = sum_ref[...] / cnt_safe
        cT = cen_ref[...]
        dot = jnp.sum(mean * cT, axis=0, keepdims=True)
        n1 = jnp.maximum(jnp.sqrt(jnp.sum(mean * mean, axis=0, keepdims=True)), EPS)
        n2 = jnp.maximum(jnp.sqrt(jnp.sum(cT * cT, axis=0, keepdims=True)), EPS)
        cos = jnp.where(present, dot / (n1 * n2), 0.0)
        npres = jnp.sum(present.astype(jnp.float32))
        loss = 1.0 - jnp.sum(cos) / npres
        out_ref[...] = loss[None, None]

    return pl.pallas_call(
        body,
        out_shape=jax.ShapeDtypeStruct((1, 1), jnp.float32),
    )(sum_pad, cnt_pad, cenT_pad)


def kernel(feature, label, centers):
    feat3 = feature.reshape(BATCH, CHANNELS, HW)
    if label.ndim == 4:
        label = label[:, 0]
    sum_pad, cnt_pad = _sc_segment_sums(label, feat3)
    cenT_pad = jnp.pad(centers.T, ((0, 0), (0, K_PAD - NUM_CLASSES)))
    loss = _epilogue(sum_pad, cnt_pad, cenT_pad)
    return loss[0, 0]


# speedup vs baseline: 4.8132x; 1.0872x over previous
import functools

import jax
import jax.numpy as jnp
from jax import lax
from jax.experimental import pallas as pl
from jax.experimental.pallas import tpu as pltpu
from jax.experimental.pallas import tpu_sc as plsc

EPS = 1e-8
NUM_CLASSES = 150
CHANNELS = 192
BATCH = 8
HW = 128 * 128
N_PIX = BATCH * HW

NC = 2
NS = 16
NW = NC * NS
LANES = 16

K_PAD = 160
CPT = CHANNELS // NW
ROWS_PER_TILE = (BATCH * 128) // NS
CNT_SLICE = N_PIX // NW


def _sc_segment_sums(label, feat3):
    mesh = plsc.VectorSubcoreMesh(core_axis_name="c", subcore_axis_name="s")

    @functools.partial(
        pl.kernel,
        out_type=(
            jax.ShapeDtypeStruct((CHANNELS, K_PAD), jnp.float32),
            jax.ShapeDtypeStruct((NW, K_PAD), jnp.float32),
        ),
        mesh=mesh,
        compiler_params=pltpu.CompilerParams(use_tc_tiling_on_sc=False,
                                             needs_layout_passes=False),
        scratch_types=[
            pltpu.VMEM((32, 512), jnp.int32),
            pltpu.VMEM((ROWS_PER_TILE * 128,), jnp.int32),
            pltpu.VMEM_SHARED((N_PIX,), jnp.int32),
            pltpu.VMEM((HW,), jnp.int32),
            pltpu.VMEM((3, HW // 2), jnp.float32),
            pltpu.VMEM((3, HW // 2), jnp.float32),
            pltpu.VMEM((CPT, LANES * K_PAD), jnp.float32),
            pltpu.VMEM((CPT, K_PAD), jnp.float32),
            pltpu.VMEM((LANES * K_PAD,), jnp.float32),
            pltpu.VMEM((K_PAD,), jnp.float32),
            pltpu.SemaphoreType.DMA,
            pltpu.SemaphoreType.DMA,
        ],
    )
    def sc_main(label_hbm, feat_hbm, sum_out, cnt_out,
                row_blk, lab_stage, lab_shared, lab_tile, feat_buf0, feat_buf1,
                acc, acc_out, cnt_acc, cnt_vec, sem0, sem1):
        cid = lax.axis_index("c")
        sid = lax.axis_index("s")
        gwid = cid * NS + sid

        zeros16 = jnp.zeros((LANES,), jnp.float32)
        ones16 = jnp.ones((LANES,), jnp.float32)
        iota16 = lax.iota(jnp.int32, LANES)
        lane_off = iota16 * K_PAD

        b0 = sid // 2
        r0 = (sid % 2) * ROWS_PER_TILE
        for chunk in range(8):
            pltpu.sync_copy(label_hbm.at[b0, pl.ds((r0 + chunk * 8) * 4, 32)],
                            row_blk)

            @plsc.parallel_loop(0, 8, unroll=2)
            def _down(rr, chunk=chunk):
                rids = jnp.broadcast_to(rr * 4, (LANES,)).astype(jnp.int32)
                for j in range(8):
                    cix = iota16 * 4 + (j * 64)
                    v = plsc.load_gather(row_blk, [rids, cix])
                    lab_stage[pl.ds((chunk * 8 + rr) * 128 + j * LANES,
                                    LANES)] = v + lane_off

        pltpu.sync_copy(
            lab_stage, lab_shared.at[pl.ds(sid * (ROWS_PER_TILE * 128),
                                           ROWS_PER_TILE * 128)])
        plsc.subcore_barrier()

        @pl.loop(0, (LANES * K_PAD) // LANES, unroll=8)
        def _zc(i):
            cnt_acc[pl.ds(i * LANES, LANES)] = zeros16

        pltpu.sync_copy(lab_shared.at[pl.ds(gwid * CNT_SLICE, CNT_SLICE)],
                        lab_tile.at[pl.ds(0, CNT_SLICE)])

        @plsc.parallel_loop(0, CNT_SLICE // LANES, unroll=8)
        def _cnt(i):
            ix = lab_tile[pl.ds(i * LANES, LANES)]
            plsc.addupdate_scatter(cnt_acc, [ix], ones16)

        @pl.loop(0, K_PAD // LANES)
        def _credu(i):
            tot = zeros16
            for l in range(LANES):
                tot = tot + cnt_acc[pl.ds(l * K_PAD + i * LANES, LANES)]
            cnt_vec[pl.ds(i * LANES, LANES)] = tot

        pltpu.sync_copy(cnt_vec, cnt_out.at[gwid])

        ch_base = gwid * CPT
        for cl in range(CPT):
            accl = acc.at[cl]

            @pl.loop(0, (LANES * K_PAD) // LANES, unroll=8)
            def _za(i, accl=accl):
                accl[pl.ds(i * LANES, LANES)] = zeros16

        feat_bufs = (feat_buf0, feat_buf1)
        sems = (sem0, sem1)
        HHW = HW // 2
        n_passes = BATCH * 2 * 2

        def _decode(p):
            b, rest = divmod(p, 4)
            t, h = divmod(rest, 2)
            return b, t, h

        def _start(p):
            b, t, h = _decode(p)
            fb, sem = feat_bufs[p % 2], sems[p % 2]
            return [
                pltpu.async_copy(
                    feat_hbm.at[b, ch_base + t * 3 + j, pl.ds(h * HHW, HHW)],
                    fb.at[j], sem)
                for j in range(3)
            ]

        descs = [_start(0), None]
        for p in range(n_passes):
            b, t, h = _decode(p)
            if t == 0 and h == 0:
                pltpu.sync_copy(lab_shared.at[pl.ds(b * HW, HW)], lab_tile)
            if p + 1 < n_passes:
                descs[(p + 1) % 2] = _start(p + 1)
            for d in descs[p % 2]:
                d.wait()
            fb = feat_bufs[p % 2]
            acc0, acc1, acc2 = (acc.at[t * 3], acc.at[t * 3 + 1],
                                acc.at[t * 3 + 2])
            lab_base = h * HHW

            @plsc.parallel_loop(0, HHW // LANES, unroll=4)
            def _seg(i, acc0=acc0, acc1=acc1, acc2=acc2, fb=fb,
                     lab_base=lab_base):
                ix = lab_tile[pl.ds(lab_base + i * LANES, LANES)]
                v0 = fb[0, pl.ds(i * LANES, LANES)]
                v1 = fb[1, pl.ds(i * LANES, LANES)]
                v2 = fb[2, pl.ds(i * LANES, LANES)]
                plsc.addupdate_scatter(acc0, [ix], v0)
                plsc.addupdate_scatter(acc1, [ix], v1)
                plsc.addupdate_scatter(acc2, [ix], v2)

        for cl in range(CPT):
            accl = acc.at[cl]
            outl = acc_out.at[cl]

            @pl.loop(0, K_PAD // LANES)
            def _aredu(i, accl=accl, outl=outl):
                tot = zeros16
                for l in range(LANES):
                    tot = tot + accl[pl.ds(l * K_PAD + i * LANES, LANES)]
                outl[pl.ds(i * LANES, LANES)] = tot

        pltpu.sync_copy(acc_out, sum_out.at[pl.ds(ch_base, CPT)])

    return sc_main(label, feat3)


def _epilogue(sum_pad, cnt_pad, cenT_pad):
    def body(sum_ref, cnt_ref, cen_ref, out_ref):
        cnt = jnp.sum(cnt_ref[...], axis=0, keepdims=True)
        present = cnt > 0.0
        cnt_safe = jnp.where(present, cnt, 1.0)
        mean = sum_ref[...] / cnt_safe
        cT = cen_ref[...]
        dot = jnp.sum(mean * cT, axis=0, keepdims=True)
        n1 = jnp.maximum(jnp.sqrt(jnp.sum(mean * mean, axis=0, keepdims=True)), EPS)
        n2 = jnp.maximum(jnp.sqrt(jnp.sum(cT * cT, axis=0, keepdims=True)), EPS)
        cos = jnp.where(present, dot / (n1 * n2), 0.0)
        npres = jnp.sum(present.astype(jnp.float32))
        loss = 1.0 - jnp.sum(cos) / npres
        out_ref[...] = loss[None, None]

    return pl.pallas_call(
        body,
        out_shape=jax.ShapeDtypeStruct((1, 1), jnp.float32),
    )(sum_pad, cnt_pad, cenT_pad)


def kernel(feature, label, centers):
    feat3 = feature.reshape(BATCH, CHANNELS, HW)
    if label.ndim == 4:
        label = label[:, 0]
    sum_pad, cnt_pad = _sc_segment_sums(label, feat3)
    cenT_pad = jnp.pad(centers.T, ((0, 0), (0, K_PAD - NUM_CLASSES)))
    loss = _epilogue(sum_pad, cnt_pad, cenT_pad)
    return loss[0, 0]


# speedup vs baseline: 5.1537x; 1.0707x over previous
import functools

import jax
import jax.numpy as jnp
from jax import lax
from jax.experimental import pallas as pl
from jax.experimental.pallas import tpu as pltpu
from jax.experimental.pallas import tpu_sc as plsc

EPS = 1e-8
NUM_CLASSES = 150
CHANNELS = 192
BATCH = 8
HW = 128 * 128
N_PIX = BATCH * HW

NC = 2
NS = 16
NW = NC * NS
LANES = 16

K_PAD = 160
CPT = CHANNELS // NW
ROWS_PER_TILE = (BATCH * 128) // NS
CNT_SLICE = N_PIX // NW
QTR = HW // 4


def _sc_segment_sums(label, feat3):
    mesh = plsc.VectorSubcoreMesh(core_axis_name="c", subcore_axis_name="s")

    @functools.partial(
        pl.kernel,
        out_type=(
            jax.ShapeDtypeStruct((CHANNELS, K_PAD), jnp.float32),
            jax.ShapeDtypeStruct((NW, K_PAD), jnp.float32),
        ),
        mesh=mesh,
        compiler_params=pltpu.CompilerParams(use_tc_tiling_on_sc=False,
                                             needs_layout_passes=False),
        scratch_types=[
            pltpu.VMEM((32, 512), jnp.int32),
            pltpu.VMEM((32, 512), jnp.int32),
            pltpu.VMEM((ROWS_PER_TILE * 128,), jnp.int32),
            pltpu.VMEM_SHARED((N_PIX,), jnp.int32),
            pltpu.VMEM((QTR,), jnp.int32),
            pltpu.VMEM((QTR,), jnp.int32),
            pltpu.VMEM((QTR,), jnp.int32),
            pltpu.VMEM((CPT, QTR), jnp.float32),
            pltpu.VMEM((CPT, QTR), jnp.float32),
            pltpu.VMEM((CPT, LANES * K_PAD), jnp.float32),
            pltpu.VMEM((CPT, K_PAD), jnp.float32),
            pltpu.VMEM((LANES * K_PAD,), jnp.float32),
            pltpu.VMEM((K_PAD,), jnp.float32),
            pltpu.SemaphoreType.DMA,
            pltpu.SemaphoreType.DMA,
            pltpu.SemaphoreType.DMA,
            pltpu.SemaphoreType.DMA,
            pltpu.SemaphoreType.DMA,
        ],
    )
    def sc_main(label_hbm, feat_hbm, sum_out, cnt_out,
                row_blk0, row_blk1, lab_stage, lab_shared, labq0, labq1,
                cnt_lab, feat_buf0, feat_buf1,
                acc, acc_out, cnt_acc, cnt_vec, sem0, sem1, lsem0, lsem1,
                rsem):
        cid = lax.axis_index("c")
        sid = lax.axis_index("s")
        gwid = cid * NS + sid

        zeros16 = jnp.zeros((LANES,), jnp.float32)
        ones16 = jnp.ones((LANES,), jnp.float32)
        iota16 = lax.iota(jnp.int32, LANES)
        lane_off = iota16 * K_PAD

        b0 = sid // 2
        r0 = (sid % 2) * ROWS_PER_TILE
        row_blks = (row_blk0, row_blk1)
        rdescs = [
            pltpu.async_copy(label_hbm.at[b0, pl.ds((r0 + half * 8) * 4, 32)],
                             row_blks[half], rsem)
            for half in range(2)
        ]
        for chunk in range(8):
            rdescs[chunk % 2].wait()
            rb = row_blks[chunk % 2]

            @plsc.parallel_loop(0, 8, unroll=2)
            def _down(rr, chunk=chunk, rb=rb):
                rids = jnp.broadcast_to(rr * 4, (LANES,)).astype(jnp.int32)
                for j in range(8):
                    cix = iota16 * 4 + (j * 64)
                    v = plsc.load_gather(rb, [rids, cix])
                    lab_stage[pl.ds((chunk * 8 + rr) * 128 + j * LANES,
                                    LANES)] = v + lane_off

            if chunk + 2 < 8:
                rdescs[chunk % 2] = pltpu.async_copy(
                    label_hbm.at[b0, pl.ds((r0 + (chunk + 2) * 8) * 4, 32)],
                    row_blks[chunk % 2], rsem)

        pltpu.sync_copy(
            lab_stage, lab_shared.at[pl.ds(sid * (ROWS_PER_TILE * 128),
                                           ROWS_PER_TILE * 128)])
        plsc.subcore_barrier()

        ch_base = gwid * CPT
        feat_bufs = (feat_buf0, feat_buf1)
        labqs = (labq0, labq1)
        sems = (sem0, sem1)
        n_passes = BATCH * 4

        lsems = (lsem0, lsem1)

        def _start(p):
            b, h = divmod(p, 4)
            fb, lq = feat_bufs[p % 2], labqs[p % 2]
            ds_pix = pl.ds(h * QTR, QTR)
            ds_lab = pl.ds(b * HW + h * QTR, QTR)
            return [
                pltpu.async_copy(
                    feat_hbm.at[b, pl.ds(ch_base, CPT), ds_pix],
                    fb, sems[p % 2]),
                pltpu.async_copy(lab_shared.at[ds_lab], lq, lsems[p % 2]),
            ]

        descs = [_start(0), None]

        @pl.loop(0, (LANES * K_PAD) // LANES, unroll=8)
        def _zc(i):
            cnt_acc[pl.ds(i * LANES, LANES)] = zeros16

        pltpu.sync_copy(lab_shared.at[pl.ds(gwid * CNT_SLICE, CNT_SLICE)],
                        cnt_lab)

        @plsc.parallel_loop(0, CNT_SLICE // LANES, unroll=8)
        def _cnt(i):
            ix = cnt_lab[pl.ds(i * LANES, LANES)]
            plsc.addupdate_scatter(cnt_acc, [ix], ones16)

        @pl.loop(0, K_PAD // LANES)
        def _credu(i):
            tot = zeros16
            for l in range(LANES):
                tot = tot + cnt_acc[pl.ds(l * K_PAD + i * LANES, LANES)]
            cnt_vec[pl.ds(i * LANES, LANES)] = tot

        pltpu.sync_copy(cnt_vec, cnt_out.at[gwid])

        for cl in range(CPT):
            accl = acc.at[cl]

            @pl.loop(0, (LANES * K_PAD) // LANES, unroll=8)
            def _za(i, accl=accl):
                accl[pl.ds(i * LANES, LANES)] = zeros16

        for p in range(n_passes):
            if p + 1 < n_passes:
                descs[(p + 1) % 2] = _start(p + 1)
            for d in descs[p % 2]:
                d.wait()
            fb, lq = feat_bufs[p % 2], labqs[p % 2]
            accs = [acc.at[j] for j in range(CPT)]

            @plsc.parallel_loop(0, QTR // LANES, unroll=4)
            def _seg(i, fb=fb, lq=lq, accs=accs):
                ix = lq[pl.ds(i * LANES, LANES)]
                for j in range(CPT):
                    v = fb[j, pl.ds(i * LANES, LANES)]
                    plsc.addupdate_scatter(accs[j], [ix], v)

        for cl in range(CPT):
            accl = acc.at[cl]
            outl = acc_out.at[cl]

            @pl.loop(0, K_PAD // LANES)
            def _aredu(i, accl=accl, outl=outl):
                tot = zeros16
                for l in range(LANES):
                    tot = tot + accl[pl.ds(l * K_PAD + i * LANES, LANES)]
                outl[pl.ds(i * LANES, LANES)] = tot

        pltpu.sync_copy(acc_out, sum_out.at[pl.ds(ch_base, CPT)])

    return sc_main(label, feat3)


def _epilogue(sum_pad, cnt_pad, cenT_pad):
    def body(sum_ref, cnt_ref, cen_ref, out_ref):
        cnt = jnp.sum(cnt_ref[...], axis=0, keepdims=True)
        present = cnt > 0.0
        cnt_safe = jnp.where(present, cnt, 1.0)
        mean = sum_ref[...] / cnt_safe
        cT = cen_ref[...]
        dot = jnp.sum(mean * cT, axis=0, keepdims=True)
        n1 = jnp.maximum(jnp.sqrt(jnp.sum(mean * mean, axis=0, keepdims=True)), EPS)
        n2 = jnp.maximum(jnp.sqrt(jnp.sum(cT * cT, axis=0, keepdims=True)), EPS)
        cos = jnp.where(present, dot / (n1 * n2), 0.0)
        npres = jnp.sum(present.astype(jnp.float32))
        loss = 1.0 - jnp.sum(cos) / npres
        out_ref[...] = loss[None, None]

    return pl.pallas_call(
        body,
        out_shape=jax.ShapeDtypeStruct((1, 1), jnp.float32),
    )(sum_pad, cnt_pad, cenT_pad)


def kernel(feature, label, centers):
    feat3 = feature.reshape(BATCH, CHANNELS, HW)
    if label.ndim == 4:
        label = label[:, 0]
    sum_pad, cnt_pad = _sc_segment_sums(label, feat3)
    cenT_pad = jnp.pad(centers.T, ((0, 0), (0, K_PAD - NUM_CLASSES)))
    loss = _epilogue(sum_pad, cnt_pad, cenT_pad)
    return loss[0, 0]
